# baseline (speedup 1.0000x reference)
import functools

import jax
import jax.numpy as jnp
from jax import lax
from jax.experimental import pallas as pl
from jax.experimental.pallas import tpu as pltpu
from jax.experimental.pallas import tpu_sc as plsc

_N = 10000
_E = 160000
_F_IN = 256
_H = 512

_N_PAD = 10240
_ROW_BLK = 512
_N_TILES = _N_PAD // _ROW_BLK
_E_PAD = 163840
_WSTEPS = 40
_NSC = 2
_NT = 16
_NW = _NSC * _NT
_EPW = _E_PAD // _NW
_QROWS = _N_PAD // 4
_QSTRIPE = _QROWS // _NT
_CAP = 10752
_GSTEP = 128
_HW = _H // 2
_SROWS = _CAP // 128
_AROWS = 2 * _QROWS
_ASTRIPE = _AROWS // _NT
_DSTRIPE = _N_PAD // _NT
_DUMMY_SRC = _N_PAD - 8


def _fill(ref, rows, cols, value):
    vec = jnp.full((16,), value, dtype=ref.dtype)
    cpr = cols // 16

    def body(t, _):
        r = t // cpr
        q = t % cpr
        ref[r, pl.ds(q * 16, 16)] = vec
        return 0

    lax.fori_loop(0, rows * cpr, body, 0)


def _deg_sc(dst3):
    mesh = plsc.VectorSubcoreMesh(core_axis_name="c", subcore_axis_name="s",
                                  num_cores=_NSC, num_subcores=_NT)

    @functools.partial(
        pl.kernel,
        out_type=jax.ShapeDtypeStruct((_NSC * _N_PAD, 128), jnp.float32),
        mesh=mesh,
        scratch_types=[
            pltpu.VMEM((_WSTEPS, 128), jnp.int32),
            pltpu.VMEM((128, 128), jnp.float32),
            pltpu.VMEM_SHARED((_N_PAD, 128), jnp.float32),
        ],
    )
    def body(dst_hbm, out_hbm, dst_v, ones_v, acc_sh):
        cid = lax.axis_index("c")
        sid = lax.axis_index("s")
        wid = cid * _NT + sid
        pltpu.sync_copy(dst_hbm.at[wid], dst_v)
        _fill(ones_v, 128, 128, 0.0)
        for k in range(_DSTRIPE // 128):
            pltpu.sync_copy(ones_v, acc_sh.at[pl.ds(sid * _DSTRIPE + k * 128, 128)])
        _fill(ones_v, 128, 128, 1.0)
        plsc.subcore_barrier()

        def step(j, _):
            pltpu.sync_copy(ones_v, acc_sh.at[dst_v.at[j]], add=True)
            return 0

        lax.fori_loop(0, _WSTEPS, step, 0)
        plsc.subcore_barrier()
        pltpu.sync_copy(
            acc_sh.at[pl.ds(sid * _DSTRIPE, _DSTRIPE)],
            out_hbm.at[pl.ds(cid * _N_PAD + sid * _DSTRIPE, _DSTRIPE)],
        )

    return body(dst3)


def _part_sc(src3, dst3):
    mesh = plsc.VectorSubcoreMesh(core_axis_name="c", subcore_axis_name="s",
                                  num_cores=_NSC, num_subcores=_NT)

    @functools.partial(
        pl.kernel,
        out_type=(
            jax.ShapeDtypeStruct((_NW * 4, _CAP), jnp.int32),
            jax.ShapeDtypeStruct((_NW * 4, _CAP), jnp.int32),
            jax.ShapeDtypeStruct((_NW, 16), jnp.int32),
        ),
        mesh=mesh,
        scratch_types=[
            pltpu.VMEM((_WSTEPS, 128), jnp.int32),
            pltpu.VMEM((_WSTEPS, 128), jnp.int32),
            pltpu.VMEM((_CAP,), jnp.int32),
            pltpu.VMEM((_CAP,), jnp.int32),
            pltpu.VMEM((_CAP,), jnp.int32),
            pltpu.VMEM((_CAP,), jnp.int32),
            pltpu.VMEM((_CAP,), jnp.int32),
            pltpu.VMEM((_CAP,), jnp.int32),
            pltpu.VMEM((_CAP,), jnp.int32),
            pltpu.VMEM((_CAP,), jnp.int32),
            pltpu.VMEM((16,), jnp.int32),
        ],
    )
    def body(src_hbm, dst_hbm, bsrc_hbm, bdst_hbm, cnt_hbm,
             src_v, dst_v, s0, d0, s1, d1, s2, d2, s3, d3, tmp_v):
        cid = lax.axis_index("c")
        sid = lax.axis_index("s")
        wid = cid * _NT + sid
        pltpu.sync_copy(src_hbm.at[wid], src_v)
        pltpu.sync_copy(dst_hbm.at[wid], dst_v)
        ls = (s0, s1, s2, s3)
        ld = (d0, d1, d2, d3)
        dn = lax.GatherDimensionNumbers(
            offset_dims=(), collapsed_slice_dims=(0,), start_index_map=(0,))

        def step(t, carry):
            r = t // 8
            q8 = t % 8
            d = dst_v[r, pl.ds(q8 * 16, 16)]
            s = src_v[r, pl.ds(q8 * 16, 16)]
            one = jnp.ones((16,), jnp.int32)
            zero = jnp.zeros((16,), jnp.int32)
            qv = (jnp.where(d >= _QROWS, one, zero)
                  + jnp.where(d >= 2 * _QROWS, one, zero)
                  + jnp.where(d >= 3 * _QROWS, one, zero))
            enc = jnp.where(
                qv == 0, jnp.full((16,), 1, jnp.int32),
                jnp.where(qv == 1, jnp.full((16,), 1 << 8, jnp.int32),
                          jnp.where(qv == 2, jnp.full((16,), 1 << 16, jnp.int32),
                                    jnp.full((16,), 1 << 24, jnp.int32))))
            iot = lax.iota(jnp.int32, 16)
            v = enc
            for sh in (1, 2, 4, 8):
                idx = jnp.maximum(iot - sh, zero)
                g = lax.gather(v, idx[:, None], dn, slice_sizes=(1,),
                               mode=lax.GatherScatterMode.PROMISE_IN_BOUNDS)
                v = v + jnp.where(iot >= sh, g, zero)
            pex = v - enc
            tmp_v[...] = v
            lv = tmp_v[...]
            packed = lv[15]
            m255 = jnp.full((16,), 255, jnp.int32)
            pcs = [jnp.bitwise_and(lax.shift_right_logical(packed, 8 * q), 255)
                   for q in range(4)]
            starts = [pcs[0] * 0, pcs[0], pcs[0] + pcs[1],
                      pcs[0] + pcs[1] + pcs[2]]
            offs_own = zero
            start_own = zero
            for q in range(4):
                member = qv == q
                bq = jnp.bitwise_and(lax.shift_right_logical(pex, 8 * q), m255)
                offs_own = jnp.where(member, bq, offs_own)
                start_own = jnp.where(
                    member, jnp.full((16,), starts[q], jnp.int32), start_own)
            target = start_own + offs_own

            def vperm(vec, idx):
                return lax.gather(vec, idx[:, None], dn, slice_sizes=(1,),
                                  mode=lax.GatherScatterMode.PROMISE_IN_BOUNDS)

            invp = zero
            for rot in range(16):
                idx = jnp.bitwise_and(iot + rot, 15)
                rt = vperm(target, idx)
                invp = jnp.where(rt == iot, idx, invp)
            dl_all = d - qv * _QROWS
            sorted_s = vperm(s, invp)
            sorted_d = vperm(dl_all, invp)
            half = lax.shift_right_logical(iot, 1)
            parity = jnp.bitwise_and(iot, 1)
            out = []
            for q in range(4):
                ridx = jnp.bitwise_and(
                    iot + jnp.full((16,), starts[q], jnp.int32), 15)
                rs = vperm(sorted_s, ridx)
                rd = vperm(sorted_d, ridx)
                c2 = 2 * carry[q]
                ls[q][pl.ds(c2, 16)] = vperm(rs, half) * 2 + parity
                ls[q][pl.ds(c2 + 16, 16)] = vperm(rs, half + 8) * 2 + parity
                ld[q][pl.ds(c2, 16)] = vperm(rd, half) * 2 + parity
                ld[q][pl.ds(c2 + 16, 16)] = vperm(rd, half + 8) * 2 + parity
                out.append(carry[q] + pcs[q])
            return tuple(out)

        carry = lax.fori_loop(0, _WSTEPS * 8, step, (0, 0, 0, 0))
        iot16 = lax.iota(jnp.int32, 16)
        par = jnp.bitwise_and(iot16, 1)
        sdum = jnp.full((16,), 2 * _DUMMY_SRC, jnp.int32) + par
        ddum = par
        sts = []
        for q in range(4):
            c2 = 2 * carry[q]
            for kk in range(8):
                ls[q][pl.ds(c2 + 16 * kk, 16)] = sdum
                ld[q][pl.ds(c2 + 16 * kk, 16)] = ddum
            sts.append(lax.div(c2 + 127, 128))
            pltpu.sync_copy(ls[q], bsrc_hbm.at[wid * 4 + q])
            pltpu.sync_copy(ld[q], bdst_hbm.at[wid * 4 + q])
        iot = lax.iota(jnp.int32, 16)
        stv = jnp.where(
            iot == 0, jnp.full((16,), sts[0], jnp.int32),
            jnp.where(iot == 1, jnp.full((16,), sts[1], jnp.int32),
                      jnp.where(iot == 2, jnp.full((16,), sts[2], jnp.int32),
                                jnp.where(iot == 3,
                                          jnp.full((16,), sts[3], jnp.int32),
                                          jnp.zeros((16,), jnp.int32)))))
        tmp_v[...] = stv
        pltpu.sync_copy(tmp_v, cnt_hbm.at[wid])

    return body(src3, dst3)


def _scat_sc(hpa, hpb, bsrc4, bdst4, cnts):
    mesh = plsc.VectorSubcoreMesh(core_axis_name="c", subcore_axis_name="s",
                                  num_cores=_NSC, num_subcores=_NT)

    @functools.partial(
        pl.kernel,
        out_type=(
            jax.ShapeDtypeStruct((2 * _N_PAD, 128), jnp.float32),
            jax.ShapeDtypeStruct((2 * _N_PAD, 128), jnp.float32),
        ),
        mesh=mesh,
        scratch_types=[
            pltpu.VMEM((_SROWS, 128), jnp.int32),
            pltpu.VMEM((_SROWS, 128), jnp.int32),
            pltpu.VMEM((_GSTEP, 128), jnp.float32),
            pltpu.VMEM((_GSTEP, 128), jnp.float32),
            pltpu.VMEM((16,), jnp.int32),
            pltpu.VMEM_SHARED((_AROWS, 128), jnp.float32),
            pltpu.SemaphoreType.DMA,
            pltpu.SemaphoreType.DMA,
        ],
    )
    def body(hpa_hbm, hpb_hbm, bsrc_hbm, bdst_hbm, cnt_hbm, outa_hbm, outb_hbm,
             idxs_v, idxd_v, rows0, rows1, cnt_v, acc_sh, sem0, sem1):
        cid = lax.axis_index("c")
        sid = lax.axis_index("s")

        def zero_stripe():
            _fill(rows0, _GSTEP, 128, 0.0)
            for k in range(_ASTRIPE // 128):
                pltpu.sync_copy(
                    rows0, acc_sh.at[pl.ds(sid * _ASTRIPE + k * 128, 128)])
            pltpu.sync_copy(
                rows0.at[pl.ds(0, _ASTRIPE % 128)],
                acc_sh.at[pl.ds(sid * _ASTRIPE + 256, _ASTRIPE % 128)])

        zero_stripe()

        def make_pass(table_hbm, out_hbm, q, last):
            plsc.subcore_barrier()
            for rr in range(2):
                w = sid * 2 + rr
                pltpu.sync_copy(cnt_hbm.at[w], cnt_v)
                cl = cnt_v[...]
                st = jnp.where(
                    q == 0, cl[0],
                    jnp.where(q == 1, cl[1], jnp.where(q == 2, cl[2], cl[3])))

                def wait_buf(buf, sem):
                    pltpu.make_async_copy(
                        table_hbm.at[pl.ds(0, _GSTEP)], buf, sem).wait()

                @pl.when(st > 0)
                def _():
                    pltpu.sync_copy(bsrc_hbm.at[w * 4 + q], idxs_v)
                    pltpu.sync_copy(bdst_hbm.at[w * 4 + q], idxd_v)
                    pltpu.async_copy(table_hbm.at[idxs_v.at[0]], rows0, sem0)

                    @pl.when(st > 1)
                    def _():
                        pltpu.async_copy(table_hbm.at[idxs_v.at[1]],
                                         rows1, sem1)

                    def pipe(jj, _):
                        j0 = 2 * jj
                        j1 = 2 * jj + 1
                        wait_buf(rows0, sem0)
                        pltpu.sync_copy(rows0, acc_sh.at[idxd_v.at[j0]],
                                        add=True)
                        n0 = jnp.minimum(j0 + 2, st - 1)
                        pltpu.async_copy(table_hbm.at[idxs_v.at[n0]],
                                         rows0, sem0)
                        wait_buf(rows1, sem1)
                        pltpu.sync_copy(rows1, acc_sh.at[idxd_v.at[j1]],
                                        add=True)
                        n1 = jnp.minimum(j1 + 2, st - 1)
                        pltpu.async_copy(table_hbm.at[idxs_v.at[n1]],
                                         rows1, sem1)
                        return 0

                    lax.fori_loop(0, lax.div(st, 2), pipe, 0)

                    @pl.when(lax.rem(st, 2) == 1)
                    def _():
                        wait_buf(rows0, sem0)
                        pltpu.sync_copy(rows0, acc_sh.at[idxd_v.at[st - 1]],
                                        add=True)

                    @pl.when(st > 1)
                    def _():
                        wait_buf(rows1, sem1)

                        @pl.when(lax.rem(st, 2) == 0)
                        def _():
                            wait_buf(rows0, sem0)

            plsc.subcore_barrier()
            pltpu.sync_copy(
                acc_sh.at[pl.ds(sid * _ASTRIPE, _ASTRIPE)],
                out_hbm.at[pl.ds(q * _AROWS + sid * _ASTRIPE, _ASTRIPE)],
            )
            if not last:
                zero_stripe()

        for qq in range(2):
            q = cid * 2 + qq
            make_pass(hpa_hbm, outa_hbm, q, False)
            make_pass(hpb_hbm, outb_hbm, q, qq == 1)

    return body(hpa, hpb, bsrc4, bdst4, cnts)


def _mm_first_body(x_ref, degf_ref, w_ref, hpa_ref, hpb_ref, dinv_ref):
    deg = degf_ref[0, :, 0:1] + degf_ref[1, :, 0:1] + 1.0
    dinv = lax.rsqrt(deg)
    h = jnp.dot(x_ref[...], w_ref[...], preferred_element_type=jnp.float32)
    hp = h * dinv
    hpa_ref[...] = hp[:, :_HW]
    hpb_ref[...] = hp[:, _HW:]
    dinv_ref[...] = dinv


def _mm_first(x_pad, degf, w1):
    return pl.pallas_call(
        _mm_first_body,
        grid=(_N_TILES,),
        in_specs=[
            pl.BlockSpec((_ROW_BLK, _F_IN), lambda i: (i, 0)),
            pl.BlockSpec((2, _ROW_BLK, 128), lambda i: (0, i, 0)),
            pl.BlockSpec((_F_IN, _H), lambda i: (0, 0)),
        ],
        out_specs=[
            pl.BlockSpec((_ROW_BLK, _HW), lambda i: (i, 0)),
            pl.BlockSpec((_ROW_BLK, _HW), lambda i: (i, 0)),
            pl.BlockSpec((_ROW_BLK, 1), lambda i: (i, 0)),
        ],
        out_shape=[
            jax.ShapeDtypeStruct((_N_PAD, _HW), jnp.float32),
            jax.ShapeDtypeStruct((_N_PAD, _HW), jnp.float32),
            jax.ShapeDtypeStruct((_N_PAD, 1), jnp.float32),
        ],
    )(x_pad, degf, w1)


def _mm_mid_body(aa_ref, ab_ref, ha_ref, hb_ref, dinv_ref, b_ref, w_ref,
                 hpa_ref, hpb_ref):
    i = pl.program_id(0)
    dinv = dinv_ref[...]
    za = dinv * (aa_ref[...] + ha_ref[...]) + b_ref[0, :_HW]
    zb = dinv * (ab_ref[...] + hb_ref[...]) + b_ref[0, _HW:]
    zc = jnp.concatenate([za, zb], axis=1)
    h = jnp.dot(zc, w_ref[...], preferred_element_type=jnp.float32)
    hpn = h * dinv
    rows = lax.broadcasted_iota(jnp.int32, (_ROW_BLK, 1), 0) + i * _ROW_BLK
    hpn = jnp.where(rows < _N, hpn, 0.0)
    hpa_ref[...] = hpn[:, :_HW]
    hpb_ref[...] = hpn[:, _HW:]


def _mm_mid(aa, ab, ha, hb, dinv, b, w):
    return pl.pallas_call(
        _mm_mid_body,
        grid=(_N_TILES,),
        in_specs=[
            pl.BlockSpec((_ROW_BLK, _HW), lambda i: (i, 0)),
            pl.BlockSpec((_ROW_BLK, _HW), lambda i: (i, 0)),
            pl.BlockSpec((_ROW_BLK, _HW), lambda i: (i, 0)),
            pl.BlockSpec((_ROW_BLK, _HW), lambda i: (i, 0)),
            pl.BlockSpec((_ROW_BLK, 1), lambda i: (i, 0)),
            pl.BlockSpec((1, _H), lambda i: (0, 0)),
            pl.BlockSpec((_H, _H), lambda i: (0, 0)),
        ],
        out_specs=[
            pl.BlockSpec((_ROW_BLK, _HW), lambda i: (i, 0)),
            pl.BlockSpec((_ROW_BLK, _HW), lambda i: (i, 0)),
        ],
        out_shape=[
            jax.ShapeDtypeStruct((_N_PAD, _HW), jnp.float32),
            jax.ShapeDtypeStruct((_N_PAD, _HW), jnp.float32),
        ],
    )(aa, ab, ha, hb, dinv, b, w)


def _final_body(aa_ref, ab_ref, ha_ref, hb_ref, dinv_ref, b_ref, out_ref):
    dinv = dinv_ref[...]
    out_ref[:, pl.ds(0, _HW)] = (
        dinv * (aa_ref[...] + ha_ref[...]) + b_ref[0, :_HW])
    out_ref[:, pl.ds(_HW, _HW)] = (
        dinv * (ab_ref[...] + hb_ref[...]) + b_ref[0, _HW:])


def _final(aa, ab, ha, hb, dinv, b):
    return pl.pallas_call(
        _final_body,
        grid=(_N_TILES,),
        in_specs=[
            pl.BlockSpec((_ROW_BLK, _HW), lambda i: (i, 0)),
            pl.BlockSpec((_ROW_BLK, _HW), lambda i: (i, 0)),
            pl.BlockSpec((_ROW_BLK, _HW), lambda i: (i, 0)),
            pl.BlockSpec((_ROW_BLK, _HW), lambda i: (i, 0)),
            pl.BlockSpec((_ROW_BLK, 1), lambda i: (i, 0)),
            pl.BlockSpec((1, _H), lambda i: (0, 0)),
        ],
        out_specs=pl.BlockSpec((_ROW_BLK, _H), lambda i: (i, 0)),
        out_shape=jax.ShapeDtypeStruct((_N_PAD, _H), jnp.float32),
    )(aa, ab, ha, hb, dinv, b)


def kernel(x, edge_index, W1, b1, W2, b2, W3, b3):
    npad = _E_PAD - _E
    src_p = jnp.concatenate(
        [edge_index[0], jnp.full((npad,), _DUMMY_SRC, dtype=jnp.int32)])
    dst_p = jnp.concatenate(
        [edge_index[1], jnp.full((npad,), _N, dtype=jnp.int32)])
    src3 = src_p.reshape(_NW, _WSTEPS, 128)
    dst3 = dst_p.reshape(_NW, _WSTEPS, 128)
    x_pad = jnp.pad(x, ((0, _N_PAD - _N), (0, 0)))

    degf = _deg_sc(dst3).reshape(_NSC, _N_PAD, 128)
    bsrc, bdst, cnts = _part_sc(src3, dst3)
    bsrc4 = bsrc.reshape(_NW * 4, _SROWS, 128)
    bdst4 = bdst.reshape(_NW * 4, _SROWS, 128)
    def scat(ha, hb):
        aa, ab = _scat_sc(ha.reshape(2 * _N_PAD, 128),
                          hb.reshape(2 * _N_PAD, 128), bsrc4, bdst4, cnts)
        return aa.reshape(_N_PAD, _HW), ab.reshape(_N_PAD, _HW)

    hp1a, hp1b, dinv = _mm_first(x_pad, degf, W1)
    a1a, a1b = scat(hp1a, hp1b)
    hp2a, hp2b = _mm_mid(a1a, a1b, hp1a, hp1b, dinv, b1.reshape(1, _H), W2)
    a2a, a2b = scat(hp2a, hp2b)
    hp3a, hp3b = _mm_mid(a2a, a2b, hp2a, hp2b, dinv, b2.reshape(1, _H), W3)
    a3a, a3b = scat(hp3a, hp3b)
    z = _final(a3a, a3b, hp3a, hp3b, dinv, b3.reshape(1, _H))
    return z[:_N]


# speedup vs baseline: 1.0015x; 1.0015x over previous
import functools

import jax
import jax.numpy as jnp
from jax import lax
from jax.experimental import pallas as pl
from jax.experimental.pallas import tpu as pltpu
from jax.experimental.pallas import tpu_sc as plsc

_N = 10000
_E = 160000
_F_IN = 256
_H = 512

_N_PAD = 10240
_ROW_BLK = 512
_N_TILES = _N_PAD // _ROW_BLK
_E_PAD = 163840
_WSTEPS = 40
_NSC = 2
_NT = 16
_NW = _NSC * _NT
_EPW = _E_PAD // _NW
_QROWS = _N_PAD // 4
_QSTRIPE = _QROWS // _NT
_CAP = 10752
_GSTEP = 128
_HW = _H // 2
_SROWS = _CAP // 128
_AROWS = 2 * _QROWS
_ASTRIPE = _AROWS // _NT
_DSTRIPE = _N_PAD // _NT
_DUMMY_SRC = _N_PAD - 8


def _fill(ref, rows, cols, value):
    vec = jnp.full((16,), value, dtype=ref.dtype)
    cpr = cols // 16

    def body(t, _):
        r = t // cpr
        q = t % cpr
        ref[r, pl.ds(q * 16, 16)] = vec
        return 0

    lax.fori_loop(0, rows * cpr, body, 0)


def _deg_sc(dst3):
    mesh = plsc.VectorSubcoreMesh(core_axis_name="c", subcore_axis_name="s",
                                  num_cores=_NSC, num_subcores=_NT)

    @functools.partial(
        pl.kernel,
        out_type=jax.ShapeDtypeStruct((_NSC * _N_PAD, 128), jnp.float32),
        mesh=mesh,
        scratch_types=[
            pltpu.VMEM((_WSTEPS, 128), jnp.int32),
            pltpu.VMEM((128, 128), jnp.float32),
            pltpu.VMEM_SHARED((_N_PAD, 128), jnp.float32),
        ],
    )
    def body(dst_hbm, out_hbm, dst_v, ones_v, acc_sh):
        cid = lax.axis_index("c")
        sid = lax.axis_index("s")
        wid = cid * _NT + sid
        pltpu.sync_copy(dst_hbm.at[wid], dst_v)
        _fill(ones_v, 128, 128, 0.0)
        for k in range(_DSTRIPE // 128):
            pltpu.sync_copy(ones_v, acc_sh.at[pl.ds(sid * _DSTRIPE + k * 128, 128)])
        _fill(ones_v, 128, 128, 1.0)
        plsc.subcore_barrier()

        def step(j, _):
            pltpu.sync_copy(ones_v, acc_sh.at[dst_v.at[j]], add=True)
            return 0

        lax.fori_loop(0, _WSTEPS, step, 0)
        plsc.subcore_barrier()
        pltpu.sync_copy(
            acc_sh.at[pl.ds(sid * _DSTRIPE, _DSTRIPE)],
            out_hbm.at[pl.ds(cid * _N_PAD + sid * _DSTRIPE, _DSTRIPE)],
        )

    return body(dst3)


def _part_sc(src3, dst3):
    mesh = plsc.VectorSubcoreMesh(core_axis_name="c", subcore_axis_name="s",
                                  num_cores=_NSC, num_subcores=_NT)

    @functools.partial(
        pl.kernel,
        out_type=(
            jax.ShapeDtypeStruct((_NW * 4, _CAP), jnp.int32),
            jax.ShapeDtypeStruct((_NW * 4, _CAP), jnp.int32),
            jax.ShapeDtypeStruct((_NW, 16), jnp.int32),
        ),
        mesh=mesh,
        scratch_types=[
            pltpu.VMEM((_WSTEPS, 128), jnp.int32),
            pltpu.VMEM((_WSTEPS, 128), jnp.int32),
            pltpu.VMEM((_CAP,), jnp.int32),
            pltpu.VMEM((_CAP,), jnp.int32),
            pltpu.VMEM((_CAP,), jnp.int32),
            pltpu.VMEM((_CAP,), jnp.int32),
            pltpu.VMEM((_CAP,), jnp.int32),
            pltpu.VMEM((_CAP,), jnp.int32),
            pltpu.VMEM((_CAP,), jnp.int32),
            pltpu.VMEM((_CAP,), jnp.int32),
            pltpu.VMEM((16,), jnp.int32),
        ],
    )
    def body(src_hbm, dst_hbm, bsrc_hbm, bdst_hbm, cnt_hbm,
             src_v, dst_v, s0, d0, s1, d1, s2, d2, s3, d3, tmp_v):
        cid = lax.axis_index("c")
        sid = lax.axis_index("s")
        wid = cid * _NT + sid
        pltpu.sync_copy(src_hbm.at[wid], src_v)
        pltpu.sync_copy(dst_hbm.at[wid], dst_v)
        ls = (s0, s1, s2, s3)
        ld = (d0, d1, d2, d3)
        dn = lax.GatherDimensionNumbers(
            offset_dims=(), collapsed_slice_dims=(0,), start_index_map=(0,))

        def step(t, carry):
            r = t // 8
            q8 = t % 8
            d = dst_v[r, pl.ds(q8 * 16, 16)]
            s = src_v[r, pl.ds(q8 * 16, 16)]
            one = jnp.ones((16,), jnp.int32)
            zero = jnp.zeros((16,), jnp.int32)
            qv = (jnp.where(d >= _QROWS, one, zero)
                  + jnp.where(d >= 2 * _QROWS, one, zero)
                  + jnp.where(d >= 3 * _QROWS, one, zero))
            enc = jnp.where(
                qv == 0, jnp.full((16,), 1, jnp.int32),
                jnp.where(qv == 1, jnp.full((16,), 1 << 8, jnp.int32),
                          jnp.where(qv == 2, jnp.full((16,), 1 << 16, jnp.int32),
                                    jnp.full((16,), 1 << 24, jnp.int32))))
            iot = lax.iota(jnp.int32, 16)
            v = enc
            for sh in (1, 2, 4, 8):
                idx = jnp.maximum(iot - sh, zero)
                g = lax.gather(v, idx[:, None], dn, slice_sizes=(1,),
                               mode=lax.GatherScatterMode.PROMISE_IN_BOUNDS)
                v = v + jnp.where(iot >= sh, g, zero)
            pex = v - enc
            tmp_v[...] = v
            lv = tmp_v[...]
            packed = lv[15]
            m255 = jnp.full((16,), 255, jnp.int32)
            pcs = [jnp.bitwise_and(lax.shift_right_logical(packed, 8 * q), 255)
                   for q in range(4)]
            starts = [pcs[0] * 0, pcs[0], pcs[0] + pcs[1],
                      pcs[0] + pcs[1] + pcs[2]]
            offs_own = zero
            start_own = zero
            for q in range(4):
                member = qv == q
                bq = jnp.bitwise_and(lax.shift_right_logical(pex, 8 * q), m255)
                offs_own = jnp.where(member, bq, offs_own)
                start_own = jnp.where(
                    member, jnp.full((16,), starts[q], jnp.int32), start_own)
            target = start_own + offs_own

            def vperm(vec, idx):
                return lax.gather(vec, idx[:, None], dn, slice_sizes=(1,),
                                  mode=lax.GatherScatterMode.PROMISE_IN_BOUNDS)

            invp = zero
            for rot in range(16):
                idx = jnp.bitwise_and(iot + rot, 15)
                rt = vperm(target, idx)
                invp = jnp.where(rt == iot, idx, invp)
            dl_all = d - qv * _QROWS
            sorted_s = vperm(s, invp)
            sorted_d = vperm(dl_all, invp)
            half = lax.shift_right_logical(iot, 1)
            parity = jnp.bitwise_and(iot, 1)
            out = []
            for q in range(4):
                ridx = jnp.bitwise_and(
                    iot + jnp.full((16,), starts[q], jnp.int32), 15)
                rs = vperm(sorted_s, ridx)
                rd = vperm(sorted_d, ridx)
                c2 = 2 * carry[q]
                ls[q][pl.ds(c2, 16)] = vperm(rs, half) * 2 + parity
                ls[q][pl.ds(c2 + 16, 16)] = vperm(rs, half + 8) * 2 + parity
                ld[q][pl.ds(c2, 16)] = vperm(rd, half) * 2 + parity
                ld[q][pl.ds(c2 + 16, 16)] = vperm(rd, half + 8) * 2 + parity
                out.append(carry[q] + pcs[q])
            return tuple(out)

        carry = lax.fori_loop(0, _WSTEPS * 8, step, (0, 0, 0, 0))
        iot16 = lax.iota(jnp.int32, 16)
        par = jnp.bitwise_and(iot16, 1)
        sdum = jnp.full((16,), 2 * _DUMMY_SRC, jnp.int32) + par
        sts = []
        for q in range(4):
            c2 = 2 * carry[q]
            for kk in range(8):
                ddum = jnp.full((16,), wid * 160 + kk * 16, jnp.int32) + iot16
                ls[q][pl.ds(c2 + 16 * kk, 16)] = sdum
                ld[q][pl.ds(c2 + 16 * kk, 16)] = ddum
            sts.append(lax.div(c2 + 127, 128))
            pltpu.sync_copy(ls[q], bsrc_hbm.at[wid * 4 + q])
            pltpu.sync_copy(ld[q], bdst_hbm.at[wid * 4 + q])
        iot = lax.iota(jnp.int32, 16)
        stv = jnp.where(
            iot == 0, jnp.full((16,), sts[0], jnp.int32),
            jnp.where(iot == 1, jnp.full((16,), sts[1], jnp.int32),
                      jnp.where(iot == 2, jnp.full((16,), sts[2], jnp.int32),
                                jnp.where(iot == 3,
                                          jnp.full((16,), sts[3], jnp.int32),
                                          jnp.zeros((16,), jnp.int32)))))
        tmp_v[...] = stv
        pltpu.sync_copy(tmp_v, cnt_hbm.at[wid])

    return body(src3, dst3)


def _scat_sc(hpa, hpb, bsrc4, bdst4, cnts):
    mesh = plsc.VectorSubcoreMesh(core_axis_name="c", subcore_axis_name="s",
                                  num_cores=_NSC, num_subcores=_NT)

    @functools.partial(
        pl.kernel,
        out_type=(
            jax.ShapeDtypeStruct((2 * _N_PAD, 128), jnp.float32),
            jax.ShapeDtypeStruct((2 * _N_PAD, 128), jnp.float32),
        ),
        mesh=mesh,
        scratch_types=[
            pltpu.VMEM((_SROWS, 128), jnp.int32),
            pltpu.VMEM((_SROWS, 128), jnp.int32),
            pltpu.VMEM((_GSTEP, 128), jnp.float32),
            pltpu.VMEM((_GSTEP, 128), jnp.float32),
            pltpu.VMEM((16,), jnp.int32),
            pltpu.VMEM_SHARED((_AROWS, 128), jnp.float32),
            pltpu.SemaphoreType.DMA,
            pltpu.SemaphoreType.DMA,
        ],
    )
    def body(hpa_hbm, hpb_hbm, bsrc_hbm, bdst_hbm, cnt_hbm, outa_hbm, outb_hbm,
             idxs_v, idxd_v, rows0, rows1, cnt_v, acc_sh, sem0, sem1):
        cid = lax.axis_index("c")
        sid = lax.axis_index("s")

        def zero_stripe():
            _fill(rows0, _GSTEP, 128, 0.0)
            for k in range(_ASTRIPE // 128):
                pltpu.sync_copy(
                    rows0, acc_sh.at[pl.ds(sid * _ASTRIPE + k * 128, 128)])
            pltpu.sync_copy(
                rows0.at[pl.ds(0, _ASTRIPE % 128)],
                acc_sh.at[pl.ds(sid * _ASTRIPE + 256, _ASTRIPE % 128)])

        zero_stripe()

        def make_pass(table_hbm, out_hbm, q, last):
            plsc.subcore_barrier()
            for rr in range(2):
                w = sid * 2 + rr
                pltpu.sync_copy(cnt_hbm.at[w], cnt_v)
                cl = cnt_v[...]
                st = jnp.where(
                    q == 0, cl[0],
                    jnp.where(q == 1, cl[1], jnp.where(q == 2, cl[2], cl[3])))

                def wait_buf(buf, sem):
                    pltpu.make_async_copy(
                        table_hbm.at[pl.ds(0, _GSTEP)], buf, sem).wait()

                @pl.when(st > 0)
                def _():
                    pltpu.sync_copy(bsrc_hbm.at[w * 4 + q], idxs_v)
                    pltpu.sync_copy(bdst_hbm.at[w * 4 + q], idxd_v)
                    pltpu.async_copy(table_hbm.at[idxs_v.at[0]], rows0, sem0)

                    @pl.when(st > 1)
                    def _():
                        pltpu.async_copy(table_hbm.at[idxs_v.at[1]],
                                         rows1, sem1)

                    def pipe(jj, _):
                        j0 = 2 * jj
                        j1 = 2 * jj + 1
                        wait_buf(rows0, sem0)
                        pltpu.sync_copy(rows0, acc_sh.at[idxd_v.at[j0]],
                                        add=True)
                        n0 = jnp.minimum(j0 + 2, st - 1)
                        pltpu.async_copy(table_hbm.at[idxs_v.at[n0]],
                                         rows0, sem0)
                        wait_buf(rows1, sem1)
                        pltpu.sync_copy(rows1, acc_sh.at[idxd_v.at[j1]],
                                        add=True)
                        n1 = jnp.minimum(j1 + 2, st - 1)
                        pltpu.async_copy(table_hbm.at[idxs_v.at[n1]],
                                         rows1, sem1)
                        return 0

                    lax.fori_loop(0, lax.div(st, 2), pipe, 0)

                    @pl.when(lax.rem(st, 2) == 1)
                    def _():
                        wait_buf(rows0, sem0)
                        pltpu.sync_copy(rows0, acc_sh.at[idxd_v.at[st - 1]],
                                        add=True)

                    @pl.when(st > 1)
                    def _():
                        wait_buf(rows1, sem1)

                        @pl.when(lax.rem(st, 2) == 0)
                        def _():
                            wait_buf(rows0, sem0)

            plsc.subcore_barrier()
            pltpu.sync_copy(
                acc_sh.at[pl.ds(sid * _ASTRIPE, _ASTRIPE)],
                out_hbm.at[pl.ds(q * _AROWS + sid * _ASTRIPE, _ASTRIPE)],
            )
            if not last:
                zero_stripe()

        for qq in range(2):
            q = cid * 2 + qq
            make_pass(hpa_hbm, outa_hbm, q, False)
            make_pass(hpb_hbm, outb_hbm, q, qq == 1)

    return body(hpa, hpb, bsrc4, bdst4, cnts)


def _mm_first_body(x_ref, degf_ref, w_ref, hpa_ref, hpb_ref, dinv_ref):
    deg = degf_ref[0, :, 0:1] + degf_ref[1, :, 0:1] + 1.0
    dinv = lax.rsqrt(deg)
    h = jnp.dot(x_ref[...], w_ref[...], preferred_element_type=jnp.float32)
    hp = h * dinv
    hpa_ref[...] = hp[:, :_HW]
    hpb_ref[...] = hp[:, _HW:]
    dinv_ref[...] = dinv


def _mm_first(x_pad, degf, w1):
    return pl.pallas_call(
        _mm_first_body,
        grid=(_N_TILES,),
        in_specs=[
            pl.BlockSpec((_ROW_BLK, _F_IN), lambda i: (i, 0)),
            pl.BlockSpec((2, _ROW_BLK, 128), lambda i: (0, i, 0)),
            pl.BlockSpec((_F_IN, _H), lambda i: (0, 0)),
        ],
        out_specs=[
            pl.BlockSpec((_ROW_BLK, _HW), lambda i: (i, 0)),
            pl.BlockSpec((_ROW_BLK, _HW), lambda i: (i, 0)),
            pl.BlockSpec((_ROW_BLK, 1), lambda i: (i, 0)),
        ],
        out_shape=[
            jax.ShapeDtypeStruct((_N_PAD, _HW), jnp.float32),
            jax.ShapeDtypeStruct((_N_PAD, _HW), jnp.float32),
            jax.ShapeDtypeStruct((_N_PAD, 1), jnp.float32),
        ],
    )(x_pad, degf, w1)


def _mm_mid_body(aa_ref, ab_ref, ha_ref, hb_ref, dinv_ref, b_ref, w_ref,
                 hpa_ref, hpb_ref):
    i = pl.program_id(0)
    dinv = dinv_ref[...]
    za = dinv * (aa_ref[...] + ha_ref[...]) + b_ref[0, :_HW]
    zb = dinv * (ab_ref[...] + hb_ref[...]) + b_ref[0, _HW:]
    zc = jnp.concatenate([za, zb], axis=1)
    h = jnp.dot(zc, w_ref[...], preferred_element_type=jnp.float32)
    hpn = h * dinv
    rows = lax.broadcasted_iota(jnp.int32, (_ROW_BLK, 1), 0) + i * _ROW_BLK
    hpn = jnp.where(rows < _N, hpn, 0.0)
    hpa_ref[...] = hpn[:, :_HW]
    hpb_ref[...] = hpn[:, _HW:]


def _mm_mid(aa, ab, ha, hb, dinv, b, w):
    return pl.pallas_call(
        _mm_mid_body,
        grid=(_N_TILES,),
        in_specs=[
            pl.BlockSpec((_ROW_BLK, _HW), lambda i: (i, 0)),
            pl.BlockSpec((_ROW_BLK, _HW), lambda i: (i, 0)),
            pl.BlockSpec((_ROW_BLK, _HW), lambda i: (i, 0)),
            pl.BlockSpec((_ROW_BLK, _HW), lambda i: (i, 0)),
            pl.BlockSpec((_ROW_BLK, 1), lambda i: (i, 0)),
            pl.BlockSpec((1, _H), lambda i: (0, 0)),
            pl.BlockSpec((_H, _H), lambda i: (0, 0)),
        ],
        out_specs=[
            pl.BlockSpec((_ROW_BLK, _HW), lambda i: (i, 0)),
            pl.BlockSpec((_ROW_BLK, _HW), lambda i: (i, 0)),
        ],
        out_shape=[
            jax.ShapeDtypeStruct((_N_PAD, _HW), jnp.float32),
            jax.ShapeDtypeStruct((_N_PAD, _HW), jnp.float32),
        ],
    )(aa, ab, ha, hb, dinv, b, w)


def _final_body(aa_ref, ab_ref, ha_ref, hb_ref, dinv_ref, b_ref, out_ref):
    dinv = dinv_ref[...]
    out_ref[:, pl.ds(0, _HW)] = (
        dinv * (aa_ref[...] + ha_ref[...]) + b_ref[0, :_HW])
    out_ref[:, pl.ds(_HW, _HW)] = (
        dinv * (ab_ref[...] + hb_ref[...]) + b_ref[0, _HW:])


def _final(aa, ab, ha, hb, dinv, b):
    return pl.pallas_call(
        _final_body,
        grid=(_N_TILES,),
        in_specs=[
            pl.BlockSpec((_ROW_BLK, _HW), lambda i: (i, 0)),
            pl.BlockSpec((_ROW_BLK, _HW), lambda i: (i, 0)),
            pl.BlockSpec((_ROW_BLK, _HW), lambda i: (i, 0)),
            pl.BlockSpec((_ROW_BLK, _HW), lambda i: (i, 0)),
            pl.BlockSpec((_ROW_BLK, 1), lambda i: (i, 0)),
            pl.BlockSpec((1, _H), lambda i: (0, 0)),
        ],
        out_specs=pl.BlockSpec((_ROW_BLK, _H), lambda i: (i, 0)),
        out_shape=jax.ShapeDtypeStruct((_N_PAD, _H), jnp.float32),
    )(aa, ab, ha, hb, dinv, b)


def kernel(x, edge_index, W1, b1, W2, b2, W3, b3):
    npad = _E_PAD - _E
    src_p = jnp.concatenate(
        [edge_index[0], jnp.full((npad,), _DUMMY_SRC, dtype=jnp.int32)])
    dst_p = jnp.concatenate(
        [edge_index[1],
         _N + jnp.arange(npad, dtype=jnp.int32) % (_N_PAD - _N)])
    src3 = src_p.reshape(_NW, _WSTEPS, 128)
    dst3 = dst_p.reshape(_NW, _WSTEPS, 128)
    x_pad = jnp.pad(x, ((0, _N_PAD - _N), (0, 0)))

    degf = _deg_sc(dst3).reshape(_NSC, _N_PAD, 128)
    bsrc, bdst, cnts = _part_sc(src3, dst3)
    bsrc4 = bsrc.reshape(_NW * 4, _SROWS, 128)
    bdst4 = bdst.reshape(_NW * 4, _SROWS, 128)
    def scat(ha, hb):
        aa, ab = _scat_sc(ha.reshape(2 * _N_PAD, 128),
                          hb.reshape(2 * _N_PAD, 128), bsrc4, bdst4, cnts)
        return aa.reshape(_N_PAD, _HW), ab.reshape(_N_PAD, _HW)

    hp1a, hp1b, dinv = _mm_first(x_pad, degf, W1)
    a1a, a1b = scat(hp1a, hp1b)
    hp2a, hp2b = _mm_mid(a1a, a1b, hp1a, hp1b, dinv, b1.reshape(1, _H), W2)
    a2a, a2b = scat(hp2a, hp2b)
    hp3a, hp3b = _mm_mid(a2a, a2b, hp2a, hp2b, dinv, b2.reshape(1, _H), W3)
    a3a, a3b = scat(hp3a, hp3b)
    z = _final(a3a, a3b, hp3a, hp3b, dinv, b3.reshape(1, _H))
    return z[:_N]


# speedup vs baseline: 1.9235x; 1.9205x over previous
import functools

import jax
import jax.numpy as jnp
from jax import lax
from jax.experimental import pallas as pl
from jax.experimental.pallas import tpu as pltpu
from jax.experimental.pallas import tpu_sc as plsc

_N = 10000
_E = 160000
_F_IN = 256
_H = 512

_N_PAD = 10240
_ROW_BLK = 512
_N_TILES = _N_PAD // _ROW_BLK
_E_PAD = 163840
_STEPS = 80
_HSTEPS = 40
_DEG_STEPS = 40
_NSC = 2
_NT = 16
_STRIPE = _N_PAD // _NT


def _fill_2d(ref, rows, value):
    vec = jnp.full((16,), value, dtype=jnp.float32)

    def body(t, _):
        r = t // 8
        q = t % 8
        ref[r, pl.ds(q * 16, 16)] = vec
        return 0

    lax.fori_loop(0, rows * 8, body, 0)


def _deg_sc(dst3):
    mesh = plsc.VectorSubcoreMesh(core_axis_name="c", subcore_axis_name="s", num_cores=_NSC, num_subcores=_NT)

    @functools.partial(
        pl.kernel,
        out_type=jax.ShapeDtypeStruct((_NSC * _N_PAD, 128), jnp.float32),
        mesh=mesh,
        scratch_types=[
            pltpu.VMEM((_DEG_STEPS, 128), jnp.int32),
            pltpu.VMEM((128, 128), jnp.float32),
            pltpu.VMEM_SHARED((_N_PAD, 128), jnp.float32),
        ],
    )
    def body(dst_hbm, out_hbm, dst_v, ones_v, acc_sh):
        cid = lax.axis_index("c")
        sid = lax.axis_index("s")
        wid = cid * _NT + sid
        pltpu.sync_copy(dst_hbm.at[wid], dst_v)
        _fill_2d(ones_v, 128, 0.0)
        for k in range(_STRIPE // 128):
            pltpu.sync_copy(ones_v, acc_sh.at[pl.ds(sid * _STRIPE + k * 128, 128)])
        _fill_2d(ones_v, 128, 1.0)
        plsc.subcore_barrier()

        def step(j, _):
            pltpu.sync_copy(ones_v, acc_sh.at[dst_v.at[j]], add=True)
            return 0

        lax.fori_loop(0, _DEG_STEPS, step, 0)
        plsc.subcore_barrier()
        pltpu.sync_copy(
            acc_sh.at[pl.ds(sid * _STRIPE, _STRIPE)],
            out_hbm.at[pl.ds(cid * _N_PAD + sid * _STRIPE, _STRIPE)],
        )

    return body(dst3)


def _scatter_sc(hp_flat, src4, dst4):
    mesh = plsc.VectorSubcoreMesh(core_axis_name="c", subcore_axis_name="s", num_cores=_NSC, num_subcores=_NT)

    @functools.partial(
        pl.kernel,
        out_type=jax.ShapeDtypeStruct((4 * _N_PAD, 128), jnp.float32),
        mesh=mesh,
        scratch_types=[
            pltpu.VMEM((_HSTEPS, 128), jnp.int32),
            pltpu.VMEM((_HSTEPS, 128), jnp.int32),
            pltpu.VMEM((128, 128), jnp.float32),
            pltpu.VMEM((128, 128), jnp.float32),
            pltpu.VMEM_SHARED((_N_PAD, 128), jnp.float32),
            pltpu.SemaphoreType.DMA,
            pltpu.SemaphoreType.DMA,
            pltpu.SemaphoreType.DMA,
            pltpu.SemaphoreType.DMA,
        ],
    )
    def body(hp_hbm, src_hbm, dst_hbm, out_hbm,
             src_v, dst_v, rows0, rows1, acc_sh, sem0, sem1, sems0, sems1):
        cid = lax.axis_index("c")
        sid = lax.axis_index("s")
        _fill_2d(rows0, 128, 0.0)
        for k in range(_STRIPE // 128):
            pltpu.sync_copy(rows0, acc_sh.at[pl.ds(sid * _STRIPE + k * 128, 128)])

        def add_off(t, _, off_val=0):
            r = t // 8
            q = t % 8
            offv = jnp.full((16,), off_val, dtype=jnp.int32)
            src_v[r, pl.ds(q * 16, 16)] = src_v[r, pl.ds(q * 16, 16)] + offv
            return 0

        def wait_buf(buf, sem):
            pltpu.make_async_copy(hp_hbm.at[pl.ds(0, 128)], buf, sem).wait()

        def wait_scat(buf, sem):
            pltpu.make_async_copy(buf, acc_sh.at[pl.ds(0, 128)], sem).wait()

        for cc in range(2):
            off = (cid * 2 + cc) * _N_PAD
            plsc.subcore_barrier()
            for hh in range(2):
                w = sid * 2 + hh
                pltpu.sync_copy(src_hbm.at[w], src_v)
                pltpu.sync_copy(dst_hbm.at[w], dst_v)
                lax.fori_loop(0, _HSTEPS * 8,
                              functools.partial(add_off, off_val=off), 0)
                pltpu.async_copy(hp_hbm.at[src_v.at[0]], rows0, sem0)
                pltpu.async_copy(hp_hbm.at[src_v.at[1]], rows1, sem1)

                def pipe(jj, _):
                    j0 = jj * 2
                    j1 = jj * 2 + 1
                    wait_buf(rows0, sem0)
                    pltpu.async_copy(rows0, acc_sh.at[dst_v.at[j0]], sems0,
                                     add=True)
                    wait_buf(rows1, sem1)
                    pltpu.async_copy(rows1, acc_sh.at[dst_v.at[j1]], sems1,
                                     add=True)
                    wait_scat(rows0, sems0)
                    n0 = lax.rem(j0 + 2, _HSTEPS)
                    pltpu.async_copy(hp_hbm.at[src_v.at[n0]], rows0, sem0)
                    wait_scat(rows1, sems1)
                    n1 = lax.rem(j1 + 2, _HSTEPS)
                    pltpu.async_copy(hp_hbm.at[src_v.at[n1]], rows1, sem1)
                    return 0

                lax.fori_loop(0, _HSTEPS // 2, pipe, 0)
                wait_buf(rows0, sem0)
                wait_buf(rows1, sem1)
            plsc.subcore_barrier()
            pltpu.sync_copy(
                acc_sh.at[pl.ds(sid * _STRIPE, _STRIPE)],
                out_hbm.at[pl.ds(off + sid * _STRIPE, _STRIPE)],
            )
            if cc == 0:
                _fill_2d(rows0, 128, 0.0)
                for k in range(_STRIPE // 128):
                    pltpu.sync_copy(
                        rows0, acc_sh.at[pl.ds(sid * _STRIPE + k * 128, 128)])

    return body(hp_flat, src4, dst4)


def _mm_first_body(x_ref, degf_ref, w_ref, hp_ref, dinv_ref):
    deg = degf_ref[0, :, 0:1] + degf_ref[1, :, 0:1] + 1.0
    dinv = lax.rsqrt(deg)
    h = jnp.dot(x_ref[...], w_ref[...], preferred_element_type=jnp.float32)
    hp = h * dinv
    for c in range(4):
        hp_ref[c] = hp[:, c * 128:(c + 1) * 128]
    dinv_ref[...] = dinv


def _mm_first(x_pad, degf, w1):
    return pl.pallas_call(
        _mm_first_body,
        grid=(_N_TILES,),
        in_specs=[
            pl.BlockSpec((_ROW_BLK, _F_IN), lambda i: (i, 0)),
            pl.BlockSpec((2, _ROW_BLK, 128), lambda i: (0, i, 0)),
            pl.BlockSpec((_F_IN, _H), lambda i: (0, 0)),
        ],
        out_specs=[
            pl.BlockSpec((4, _ROW_BLK, 128), lambda i: (0, i, 0)),
            pl.BlockSpec((_ROW_BLK, 1), lambda i: (i, 0)),
        ],
        out_shape=[
            jax.ShapeDtypeStruct((4, _N_PAD, 128), jnp.float32),
            jax.ShapeDtypeStruct((_N_PAD, 1), jnp.float32),
        ],
    )(x_pad, degf, w1)


def _mm_mid_body(agg_ref, hp_ref, dinv_ref, b_ref, w_ref, out_ref):
    dinv = dinv_ref[...]
    acc = jnp.zeros((_ROW_BLK, _H), dtype=jnp.float32)
    for c in range(4):
        zc = dinv * (agg_ref[c] + hp_ref[c]) + b_ref[0, c * 128:(c + 1) * 128]
        acc = acc + jnp.dot(zc, w_ref[pl.ds(c * 128, 128), :],
                            preferred_element_type=jnp.float32)
    hpn = acc * dinv
    for c in range(4):
        out_ref[c] = hpn[:, c * 128:(c + 1) * 128]


def _mm_mid(agg, hp, dinv, b, w):
    return pl.pallas_call(
        _mm_mid_body,
        grid=(_N_TILES,),
        in_specs=[
            pl.BlockSpec((4, _ROW_BLK, 128), lambda i: (0, i, 0)),
            pl.BlockSpec((4, _ROW_BLK, 128), lambda i: (0, i, 0)),
            pl.BlockSpec((_ROW_BLK, 1), lambda i: (i, 0)),
            pl.BlockSpec((1, _H), lambda i: (0, 0)),
            pl.BlockSpec((_H, _H), lambda i: (0, 0)),
        ],
        out_specs=pl.BlockSpec((4, _ROW_BLK, 128), lambda i: (0, i, 0)),
        out_shape=jax.ShapeDtypeStruct((4, _N_PAD, 128), jnp.float32),
    )(agg, hp, dinv, b, w)


def _final_body(agg_ref, hp_ref, dinv_ref, b_ref, out_ref):
    dinv = dinv_ref[...]
    for c in range(4):
        out_ref[:, pl.ds(c * 128, 128)] = (
            dinv * (agg_ref[c] + hp_ref[c]) + b_ref[0, c * 128:(c + 1) * 128])


def _final(agg, hp, dinv, b):
    return pl.pallas_call(
        _final_body,
        grid=(_N_TILES,),
        in_specs=[
            pl.BlockSpec((4, _ROW_BLK, 128), lambda i: (0, i, 0)),
            pl.BlockSpec((4, _ROW_BLK, 128), lambda i: (0, i, 0)),
            pl.BlockSpec((_ROW_BLK, 1), lambda i: (i, 0)),
            pl.BlockSpec((1, _H), lambda i: (0, 0)),
        ],
        out_specs=pl.BlockSpec((_ROW_BLK, _H), lambda i: (i, 0)),
        out_shape=jax.ShapeDtypeStruct((_N_PAD, _H), jnp.float32),
    )(agg, hp, dinv, b)


def kernel(x, edge_index, W1, b1, W2, b2, W3, b3):
    npad = _E_PAD - _E
    src_p = jnp.concatenate(
        [edge_index[0], jnp.zeros((npad,), dtype=jnp.int32)])
    dst_p = jnp.concatenate(
        [edge_index[1], jnp.full((npad,), _N, dtype=jnp.int32)])
    src3 = src_p.reshape(_NT * 2, _HSTEPS, 128)
    dst3 = dst_p.reshape(_NT * 2, _HSTEPS, 128)
    dst3_deg = dst_p.reshape(_NSC * _NT, _DEG_STEPS, 128)
    x_pad = jnp.pad(x, ((0, _N_PAD - _N), (0, 0)))

    degf = _deg_sc(dst3_deg).reshape(_NSC, _N_PAD, 128)
    hp1, dinv = _mm_first(x_pad, degf, W1)
    agg1 = _scatter_sc(hp1.reshape(4 * _N_PAD, 128), src3, dst3)
    hp2 = _mm_mid(agg1.reshape(4, _N_PAD, 128), hp1, dinv,
                  b1.reshape(1, _H), W2)
    agg2 = _scatter_sc(hp2.reshape(4 * _N_PAD, 128), src3, dst3)
    hp3 = _mm_mid(agg2.reshape(4, _N_PAD, 128), hp2, dinv,
                  b2.reshape(1, _H), W3)
    agg3 = _scatter_sc(hp3.reshape(4 * _N_PAD, 128), src3, dst3)
    z = _final(agg3.reshape(4, _N_PAD, 128), hp3, dinv, b3.reshape(1, _H))
    return z[:_N]


# speedup vs baseline: 2.0886x; 1.0858x over previous
import functools

import jax
import jax.numpy as jnp
from jax import lax
from jax.experimental import pallas as pl
from jax.experimental.pallas import tpu as pltpu
from jax.experimental.pallas import tpu_sc as plsc

_N = 10000
_E = 160000
_F_IN = 256
_H = 512

_N_PAD = 10240
_ROW_BLK = 512
_N_TILES = _N_PAD // _ROW_BLK
_E_PAD = 163840
_STEPS = 80
_HSTEPS = 40
_DEG_STEPS = 40
_NSC = 2
_NT = 16
_STRIPE = _N_PAD // _NT


def _fill_2d(ref, rows, value):
    vec = jnp.full((16,), value, dtype=jnp.float32)

    def body(t, _):
        r = t // 8
        q = t % 8
        ref[r, pl.ds(q * 16, 16)] = vec
        return 0

    lax.fori_loop(0, rows * 8, body, 0)


def _deg_sc(dst3):
    mesh = plsc.VectorSubcoreMesh(core_axis_name="c", subcore_axis_name="s", num_cores=_NSC, num_subcores=_NT)

    @functools.partial(
        pl.kernel,
        out_type=jax.ShapeDtypeStruct((_NSC * _N_PAD, 128), jnp.float32),
        mesh=mesh,
        scratch_types=[
            pltpu.VMEM((_DEG_STEPS, 128), jnp.int32),
            pltpu.VMEM((128, 128), jnp.float32),
            pltpu.VMEM_SHARED((_N_PAD, 128), jnp.float32),
        ],
    )
    def body(dst_hbm, out_hbm, dst_v, ones_v, acc_sh):
        cid = lax.axis_index("c")
        sid = lax.axis_index("s")
        wid = cid * _NT + sid
        pltpu.sync_copy(dst_hbm.at[wid], dst_v)
        _fill_2d(ones_v, 128, 0.0)
        for k in range(_STRIPE // 128):
            pltpu.sync_copy(ones_v, acc_sh.at[pl.ds(sid * _STRIPE + k * 128, 128)])
        _fill_2d(ones_v, 128, 1.0)
        plsc.subcore_barrier()

        def step(j, _):
            pltpu.sync_copy(ones_v, acc_sh.at[dst_v.at[j]], add=True)
            return 0

        lax.fori_loop(0, _DEG_STEPS, step, 0)
        plsc.subcore_barrier()
        pltpu.sync_copy(
            acc_sh.at[pl.ds(sid * _STRIPE, _STRIPE)],
            out_hbm.at[pl.ds(cid * _N_PAD + sid * _STRIPE, _STRIPE)],
        )

    return body(dst3)


def _scatter_sc(hp_flat, src4, dst4):
    mesh = plsc.VectorSubcoreMesh(core_axis_name="c", subcore_axis_name="s", num_cores=_NSC, num_subcores=_NT)

    @functools.partial(
        pl.kernel,
        out_type=jax.ShapeDtypeStruct((4 * _N_PAD, 128), jnp.float32),
        mesh=mesh,
        scratch_types=[
            pltpu.VMEM((_HSTEPS, 128), jnp.int32),
            pltpu.VMEM((_HSTEPS, 128), jnp.int32),
            pltpu.VMEM((128, 128), jnp.float32),
            pltpu.VMEM((128, 128), jnp.float32),
            pltpu.VMEM_SHARED((_N_PAD, 128), jnp.float32),
            pltpu.SemaphoreType.DMA,
            pltpu.SemaphoreType.DMA,
        ],
    )
    def body(hp_hbm, src_hbm, dst_hbm, out_hbm,
             src_v, dst_v, rows0, rows1, acc_sh, sem0, sem1):
        cid = lax.axis_index("c")
        sid = lax.axis_index("s")
        _fill_2d(rows0, 128, 0.0)
        for k in range(_STRIPE // 128):
            pltpu.sync_copy(rows0, acc_sh.at[pl.ds(sid * _STRIPE + k * 128, 128)])

        def add_off(t, _, off_val=0):
            r = t // 8
            q = t % 8
            offv = jnp.full((16,), off_val, dtype=jnp.int32)
            src_v[r, pl.ds(q * 16, 16)] = src_v[r, pl.ds(q * 16, 16)] + offv
            return 0

        def wait_buf(buf, sem):
            pltpu.make_async_copy(hp_hbm.at[pl.ds(0, 128)], buf, sem).wait()

        for cc in range(2):
            off = (cid * 2 + cc) * _N_PAD
            plsc.subcore_barrier()
            for hh in range(2):
                w = sid * 2 + hh
                pltpu.sync_copy(src_hbm.at[w], src_v)
                pltpu.sync_copy(dst_hbm.at[w], dst_v)
                lax.fori_loop(0, _HSTEPS * 8,
                              functools.partial(add_off, off_val=off), 0)
                pltpu.async_copy(hp_hbm.at[src_v.at[0]], rows0, sem0)

                def pipe(jj, _):
                    j0 = jj * 2
                    j1 = jj * 2 + 1
                    pltpu.async_copy(hp_hbm.at[src_v.at[j1]], rows1, sem1)
                    wait_buf(rows0, sem0)
                    pltpu.sync_copy(rows0, acc_sh.at[dst_v.at[j0]], add=True)
                    nxt = lax.rem(j0 + 2, _HSTEPS)
                    pltpu.async_copy(hp_hbm.at[src_v.at[nxt]], rows0, sem0)
                    wait_buf(rows1, sem1)
                    pltpu.sync_copy(rows1, acc_sh.at[dst_v.at[j1]], add=True)
                    return 0

                lax.fori_loop(0, _HSTEPS // 2, pipe, 0)
                wait_buf(rows0, sem0)
            plsc.subcore_barrier()
            pltpu.sync_copy(
                acc_sh.at[pl.ds(sid * _STRIPE, _STRIPE)],
                out_hbm.at[pl.ds(off + sid * _STRIPE, _STRIPE)],
            )
            if cc == 0:
                _fill_2d(rows0, 128, 0.0)
                for k in range(_STRIPE // 128):
                    pltpu.sync_copy(
                        rows0, acc_sh.at[pl.ds(sid * _STRIPE + k * 128, 128)])

    return body(hp_flat, src4, dst4)


def _mm_first_body(x_ref, degf_ref, w_ref, hp_ref, dinv_ref):
    deg = degf_ref[0, :, 0:1] + degf_ref[1, :, 0:1] + 1.0
    dinv = lax.rsqrt(deg)
    h = jnp.dot(x_ref[...], w_ref[...], preferred_element_type=jnp.float32)
    hp = h * dinv
    for c in range(4):
        hp_ref[c] = hp[:, c * 128:(c + 1) * 128]
    dinv_ref[...] = dinv


def _mm_first(x_pad, degf, w1):
    return pl.pallas_call(
        _mm_first_body,
        grid=(_N_TILES,),
        in_specs=[
            pl.BlockSpec((_ROW_BLK, _F_IN), lambda i: (i, 0)),
            pl.BlockSpec((2, _ROW_BLK, 128), lambda i: (0, i, 0)),
            pl.BlockSpec((_F_IN, _H), lambda i: (0, 0)),
        ],
        out_specs=[
            pl.BlockSpec((4, _ROW_BLK, 128), lambda i: (0, i, 0)),
            pl.BlockSpec((_ROW_BLK, 1), lambda i: (i, 0)),
        ],
        out_shape=[
            jax.ShapeDtypeStruct((4, _N_PAD, 128), jnp.float32),
            jax.ShapeDtypeStruct((_N_PAD, 1), jnp.float32),
        ],
    )(x_pad, degf, w1)


def _mm_mid_body(agg_ref, hp_ref, dinv_ref, b_ref, w_ref, out_ref):
    dinv = dinv_ref[...]
    acc = jnp.zeros((_ROW_BLK, _H), dtype=jnp.float32)
    for c in range(4):
        zc = dinv * (agg_ref[c] + hp_ref[c]) + b_ref[0, c * 128:(c + 1) * 128]
        acc = acc + jnp.dot(zc, w_ref[pl.ds(c * 128, 128), :],
                            preferred_element_type=jnp.float32)
    hpn = acc * dinv
    for c in range(4):
        out_ref[c] = hpn[:, c * 128:(c + 1) * 128]


def _mm_mid(agg, hp, dinv, b, w):
    return pl.pallas_call(
        _mm_mid_body,
        grid=(_N_TILES,),
        in_specs=[
            pl.BlockSpec((4, _ROW_BLK, 128), lambda i: (0, i, 0)),
            pl.BlockSpec((4, _ROW_BLK, 128), lambda i: (0, i, 0)),
            pl.BlockSpec((_ROW_BLK, 1), lambda i: (i, 0)),
            pl.BlockSpec((1, _H), lambda i: (0, 0)),
            pl.BlockSpec((_H, _H), lambda i: (0, 0)),
        ],
        out_specs=pl.BlockSpec((4, _ROW_BLK, 128), lambda i: (0, i, 0)),
        out_shape=jax.ShapeDtypeStruct((4, _N_PAD, 128), jnp.float32),
    )(agg, hp, dinv, b, w)


def _final_body(agg_ref, hp_ref, dinv_ref, b_ref, out_ref):
    dinv = dinv_ref[...]
    for c in range(4):
        out_ref[:, pl.ds(c * 128, 128)] = (
            dinv * (agg_ref[c] + hp_ref[c]) + b_ref[0, c * 128:(c + 1) * 128])


def _final(agg, hp, dinv, b):
    return pl.pallas_call(
        _final_body,
        grid=(_N_TILES,),
        in_specs=[
            pl.BlockSpec((4, _ROW_BLK, 128), lambda i: (0, i, 0)),
            pl.BlockSpec((4, _ROW_BLK, 128), lambda i: (0, i, 0)),
            pl.BlockSpec((_ROW_BLK, 1), lambda i: (i, 0)),
            pl.BlockSpec((1, _H), lambda i: (0, 0)),
        ],
        out_specs=pl.BlockSpec((_ROW_BLK, _H), lambda i: (i, 0)),
        out_shape=jax.ShapeDtypeStruct((_N_PAD, _H), jnp.float32),
    )(agg, hp, dinv, b)


def kernel(x, edge_index, W1, b1, W2, b2, W3, b3):
    npad = _E_PAD - _E
    src_p = jnp.concatenate(
        [edge_index[0], jnp.zeros((npad,), dtype=jnp.int32)])
    dst_p = jnp.concatenate(
        [edge_index[1], jnp.full((npad,), _N, dtype=jnp.int32)])
    src3 = src_p.reshape(_NT * 2, _HSTEPS, 128)
    dst3 = dst_p.reshape(_NT * 2, _HSTEPS, 128)
    dst3_deg = dst_p.reshape(_NSC * _NT, _DEG_STEPS, 128)
    x_pad = jnp.pad(x, ((0, _N_PAD - _N), (0, 0)))

    degf = _deg_sc(dst3_deg).reshape(_NSC, _N_PAD, 128)
    hp1, dinv = _mm_first(x_pad, degf, W1)
    agg1 = _scatter_sc(hp1.reshape(4 * _N_PAD, 128), src3, dst3)
    hp2 = _mm_mid(agg1.reshape(4, _N_PAD, 128), hp1, dinv,
                  b1.reshape(1, _H), W2)
    agg2 = _scatter_sc(hp2.reshape(4 * _N_PAD, 128), src3, dst3)
    hp3 = _mm_mid(agg2.reshape(4, _N_PAD, 128), hp2, dinv,
                  b2.reshape(1, _H), W3)
    agg3 = _scatter_sc(hp3.reshape(4 * _N_PAD, 128), src3, dst3)
    z = _final(agg3.reshape(4, _N_PAD, 128), hp3, dinv, b3.reshape(1, _H))
    return z[:_N]
